# baseline (speedup 1.0000x reference)
import functools

import jax
import jax.numpy as jnp
from jax import lax
from jax.experimental import pallas as pl
from jax.experimental.pallas import tpu as pltpu
from jax.experimental.pallas import tpu_sc as plsc

NC = 2
NS = 16
NW = NC * NS
L = 16


@jax.jit
def _sc_store_gather(mem, idx, val, read_idx):
    M, D = mem.shape
    B = idx.shape[0]
    RANGE = ((M + NS - 1) // NS + 8 * L - 1) // (8 * L) * (8 * L)
    RPW = B // NW
    CH = 16
    NB = RPW // CH
    NR = 4
    NGW = B // L
    U = 8
    GPB = NGW // NB
    NGR = RPW // L
    HCAP = RPW + 2 * L

    mesh = plsc.VectorSubcoreMesh(core_axis_name="c", subcore_axis_name="s",
                                  num_cores=NC, num_subcores=NS)

    @functools.partial(
        pl.kernel,
        out_type=jax.ShapeDtypeStruct((B, D), jnp.float32),
        mesh=mesh,
        compiler_params=pltpu.CompilerParams(needs_layout_passes=False),
        scratch_types=[
            pltpu.VMEM((B,), jnp.int32),
            pltpu.VMEM((RANGE,), jnp.int32),
            pltpu.VMEM_SHARED((NS * RANGE,), jnp.int32),
            pltpu.VMEM((RPW,), jnp.int32),
            pltpu.VMEM((RPW,), jnp.int32),
            pltpu.VMEM((HCAP,), jnp.int32),
            pltpu.VMEM((HCAP,), jnp.int32),
            pltpu.VMEM((NR, CH, D), jnp.float32),
            pltpu.VMEM((L, D), jnp.float32),
            pltpu.SemaphoreType.DMA,
            pltpu.SemaphoreType.DMA,
        ],
    )
    def body(mem_h, idx_h, val_h, ridx_h, out_h,
             idx_v, marker_v, shared_m, ridx_v, g_v, hsrc_v, hdst_v,
             rowbuf, hitbuf, sem_g, sem_s):
        c = lax.axis_index("c")
        s = lax.axis_index("s")
        wid = c * NS + s
        base = wid * RPW
        lanes = lax.iota(jnp.int32, L)
        lo = s * RANGE

        with jax.named_scope("ph_stage"):
            pltpu.sync_copy(ridx_h.at[pl.ds(base, RPW)], ridx_v)

        def gather_chunk(t):
            return pltpu.async_copy(
                mem_h.at[ridx_v.at[pl.ds(t * CH, CH)]],
                rowbuf.at[t % NR], sem_g)

        def scatter_chunk(t):
            return pltpu.async_copy(
                rowbuf.at[t % NR], out_h.at[pl.ds(base + t * CH, CH)], sem_s)

        gd = [None] * NB
        gd[0] = gather_chunk(0)
        gd[1] = gather_chunk(1)

        idx_dma = pltpu.async_copy(idx_h, idx_v, sem_s)

        with jax.named_scope("ph_init"):
            neg1 = jnp.full((L,), -1, jnp.int32)

            def init_body(i, carry):
                for k in range(8):
                    marker_v[pl.ds(i * (8 * L) + k * L, L)] = neg1
                return carry

            lax.fori_loop(0, RANGE // (8 * L), init_body, 0)
        idx_dma.wait()

        lov = jnp.zeros((L,), jnp.int32) + lo
        rangev = jnp.full((L,), RANGE, jnp.uint32)
        zerov = jnp.zeros((L,), jnp.int32)

        def scan_group(off, jv):
            v = idx_v[pl.ds(off, L)]
            loc = v - lov
            inr = plsc.bitcast(loc, jnp.uint32) < rangev
            _, last = plsc.scan_count(v)
            locs = jnp.where(inr, loc, zerov)
            plsc.store_scatter(marker_v, [locs], jv, mask=inr & last)
            return jv + L

        sd = [None] * NB
        jv = lanes
        scan_scope = jax.named_scope("ph_scan")
        scan_scope.__enter__()
        for t in range(NB):
            def blk_body(i, jv, t=t):
                for k in range(U):
                    jv = scan_group(t * GPB * L + i * U * L + k * L, jv)
                return jv

            jv = lax.fori_loop(0, GPB // U, blk_body, jv)
            with jax.named_scope("ph_dwait"):
                gd[t].wait()
                if t >= 2:
                    sd[t - 2].wait()
                if t + 2 < NB:
                    gd[t + 2] = gather_chunk(t + 2)
                sd[t] = scatter_chunk(t)
        sd[NB - 2].wait()
        sd[NB - 1].wait()
        scan_scope.__exit__(None, None, None)

        with jax.named_scope("ph_publish"):
            pltpu.sync_copy(marker_v, shared_m.at[pl.ds(s * RANGE, RANGE)])
            plsc.subcore_barrier()

        with jax.named_scope("ph_lookup"):
            mk = [pltpu.async_copy(
                      shared_m.at[ridx_v.at[pl.ds(k * 128, 128)]],
                      g_v.at[pl.ds(k * 128, 128)], sem_g)
                  for k in range(RPW // 128)]
            for d in mk:
                d.wait()

        def hit_body(g, carry):
            nh, psrc, pdst = carry
            gv = g_v[pl.ds(g * L, L)]
            hit = gv >= 0
            dstv = base + g * L + lanes
            plsc.store_compressed(hsrc_v.at[pl.ds(nh, L)], gv, mask=hit)
            plsc.store_compressed(hdst_v.at[pl.ds(nh, L)], dstv, mask=hit)
            cnt = plsc.all_reduce_population_count(hit)[0]
            dmax = jnp.max(jnp.where(hit, dstv, -1))
            smax = jnp.max(jnp.where(dstv == dmax, gv, -1))
            psrc = jnp.where(cnt > 0, smax, psrc)
            pdst = jnp.where(cnt > 0, dmax, pdst)
            return nh + cnt, psrc, pdst

        with jax.named_scope("ph_hits"):
            nh, psrc, pdst = lax.fori_loop(0, NGR, hit_body, (0, 0, base))

        zeros = jnp.zeros((L,), jnp.int32)
        hsrc_v[pl.ds(nh, L)] = zeros + psrc
        hdst_v[pl.ds(nh, L)] = zeros + pdst

        def fix_body(t, carry):
            sv = hsrc_v[pl.ds(t * L, L)]
            dv = hdst_v[pl.ds(t * L, L)]
            pltpu.async_copy(val_h.at[sv], hitbuf, sem_g).wait()
            pltpu.async_copy(hitbuf, out_h.at[dv], sem_s).wait()
            return carry

        with jax.named_scope("ph_patch"):
            lax.fori_loop(0, (nh + L - 1) // L, fix_body, 0)

    return body(mem, idx, val, read_idx)


def kernel(mem, idx, val, read_idx):
    return _sc_store_gather(mem, idx, val, read_idx)


# speedup vs baseline: 1.0472x; 1.0472x over previous
import functools

import jax
import jax.numpy as jnp
from jax import lax
from jax.experimental import pallas as pl
from jax.experimental.pallas import tpu as pltpu
from jax.experimental.pallas import tpu_sc as plsc

NC = 2
NS = 16
NW = NC * NS
L = 16


@jax.jit
def _sc_store_gather(mem, idx, val, read_idx):
    M, D = mem.shape
    B = idx.shape[0]
    RANGE = ((M + NS - 1) // NS + 8 * L - 1) // (8 * L) * (8 * L)
    RPW = B // NW
    CH = 16
    NB = RPW // CH
    NR = 4
    NGW = B // L
    U = 4
    GPB = NGW // NB
    NGR = RPW // L
    HCAP = RPW + 2 * L

    mesh = plsc.VectorSubcoreMesh(core_axis_name="c", subcore_axis_name="s",
                                  num_cores=NC, num_subcores=NS)

    @functools.partial(
        pl.kernel,
        out_type=jax.ShapeDtypeStruct((B, D), jnp.float32),
        mesh=mesh,
        compiler_params=pltpu.CompilerParams(needs_layout_passes=False),
        scratch_types=[
            pltpu.VMEM((B,), jnp.int32),
            pltpu.VMEM((RANGE,), jnp.int32),
            pltpu.VMEM_SHARED((NS * RANGE,), jnp.int32),
            pltpu.VMEM((RPW,), jnp.int32),
            pltpu.VMEM((RPW,), jnp.int32),
            pltpu.VMEM((HCAP,), jnp.int32),
            pltpu.VMEM((HCAP,), jnp.int32),
            pltpu.VMEM((NR, CH, D), jnp.float32),
            pltpu.VMEM((L, D), jnp.float32),
            pltpu.SemaphoreType.DMA,
            pltpu.SemaphoreType.DMA,
        ],
    )
    def body(mem_h, idx_h, val_h, ridx_h, out_h,
             idx_v, marker_v, shared_m, ridx_v, g_v, hsrc_v, hdst_v,
             rowbuf, hitbuf, sem_g, sem_s):
        c = lax.axis_index("c")
        s = lax.axis_index("s")
        wid = c * NS + s
        base = wid * RPW
        lanes = lax.iota(jnp.int32, L)
        lo = s * RANGE

        with jax.named_scope("ph_stage"):
            pltpu.sync_copy(ridx_h.at[pl.ds(base, RPW)], ridx_v)

        def gather_chunk(t):
            return pltpu.async_copy(
                mem_h.at[ridx_v.at[pl.ds(t * CH, CH)]],
                rowbuf.at[t % NR], sem_g)

        def scatter_chunk(t):
            return pltpu.async_copy(
                rowbuf.at[t % NR], out_h.at[pl.ds(base + t * CH, CH)], sem_s)

        gd = [None] * NB
        gd[0] = gather_chunk(0)
        gd[1] = gather_chunk(1)

        idx_dma = pltpu.async_copy(idx_h, idx_v, sem_s)

        with jax.named_scope("ph_init"):
            neg1 = jnp.full((L,), -1, jnp.int32)

            def init_body(i, carry):
                for k in range(8):
                    marker_v[pl.ds(i * (8 * L) + k * L, L)] = neg1
                return carry

            lax.fori_loop(0, RANGE // (8 * L), init_body, 0)
        idx_dma.wait()

        lov = jnp.zeros((L,), jnp.int32) + lo
        rangev = jnp.full((L,), RANGE, jnp.uint32)
        zerov = jnp.zeros((L,), jnp.int32)

        def scan_group(off, jv):
            v = idx_v[pl.ds(off, L)]
            loc = v - lov
            inr = plsc.bitcast(loc, jnp.uint32) < rangev
            _, last = plsc.scan_count(v)
            locs = jnp.where(inr, loc, zerov)
            plsc.store_scatter(marker_v, [locs], jv, mask=inr & last)
            return jv + L

        sd = [None] * NB
        jv = lanes
        scan_scope = jax.named_scope("ph_scan")
        scan_scope.__enter__()
        for t in range(NB):
            def blk_body(i, jv, t=t):
                for k in range(U):
                    jv = scan_group(t * GPB * L + i * U * L + k * L, jv)
                return jv

            jv = lax.fori_loop(0, GPB // U, blk_body, jv)
            with jax.named_scope("ph_dwait"):
                gd[t].wait()
                if t >= 2:
                    sd[t - 2].wait()
                if t + 2 < NB:
                    gd[t + 2] = gather_chunk(t + 2)
                sd[t] = scatter_chunk(t)
        sd[NB - 2].wait()
        sd[NB - 1].wait()
        scan_scope.__exit__(None, None, None)

        with jax.named_scope("ph_publish"):
            pltpu.sync_copy(marker_v, shared_m.at[pl.ds(s * RANGE, RANGE)])
            plsc.subcore_barrier()

        with jax.named_scope("ph_lookup"):
            mk = [pltpu.async_copy(
                      shared_m.at[ridx_v.at[pl.ds(k * 128, 128)]],
                      g_v.at[pl.ds(k * 128, 128)], sem_g)
                  for k in range(RPW // 128)]
            for d in mk:
                d.wait()

        def hit_body(g, carry):
            nh, psrc, pdst = carry
            gv = g_v[pl.ds(g * L, L)]
            hit = gv >= 0
            dstv = base + g * L + lanes
            plsc.store_compressed(hsrc_v.at[pl.ds(nh, L)], gv, mask=hit)
            plsc.store_compressed(hdst_v.at[pl.ds(nh, L)], dstv, mask=hit)
            cnt = plsc.all_reduce_population_count(hit)[0]
            dmax = jnp.max(jnp.where(hit, dstv, -1))
            smax = jnp.max(jnp.where(dstv == dmax, gv, -1))
            psrc = jnp.where(cnt > 0, smax, psrc)
            pdst = jnp.where(cnt > 0, dmax, pdst)
            return nh + cnt, psrc, pdst

        with jax.named_scope("ph_hits"):
            nh, psrc, pdst = lax.fori_loop(0, NGR, hit_body, (0, 0, base))

        zeros = jnp.zeros((L,), jnp.int32)
        hsrc_v[pl.ds(nh, L)] = zeros + psrc
        hdst_v[pl.ds(nh, L)] = zeros + pdst

        def fix_body(t, carry):
            sv = hsrc_v[pl.ds(t * L, L)]
            dv = hdst_v[pl.ds(t * L, L)]
            pltpu.async_copy(val_h.at[sv], hitbuf, sem_g).wait()
            pltpu.async_copy(hitbuf, out_h.at[dv], sem_s).wait()
            return carry

        with jax.named_scope("ph_patch"):
            lax.fori_loop(0, (nh + L - 1) // L, fix_body, 0)

    return body(mem, idx, val, read_idx)


def kernel(mem, idx, val, read_idx):
    return _sc_store_gather(mem, idx, val, read_idx)
